# baseline (speedup 1.0000x reference)
import jax
import jax.numpy as jnp
from jax.experimental import pallas as pl
from jax.experimental.pallas import tpu as pltpu

_B = 16
_T = 512
_DE = 512
_DP = 320
_DJ = 320
_V = 1024
_BLANK = 0
_TCH = 128
_MB = 1024

_f32 = jnp.float32
_bf16 = jnp.bfloat16


def _proj_kernel(enc_ref, wenc_ref, bj_ref, out_ref):
    out_ref[...] = (jnp.dot(enc_ref[...], wenc_ref[...],
                            preferred_element_type=_f32)
                    + bj_ref[...])


def _xsplit(x):
    hi = x.astype(_bf16)
    lo = (x - hi.astype(_f32)).astype(_bf16)
    return hi, lo


def _dot3(xhi, xlo, whi_ref, wlo_ref):
    return (jnp.dot(xhi, whi_ref[...], preferred_element_type=_f32)
            + jnp.dot(xlo, whi_ref[...], preferred_element_type=_f32)
            + jnp.dot(xhi, wlo_ref[...], preferred_element_type=_f32))


def _decode_kernel(encp_ref, lens_ref, ehi_ref, elo_ref,
                   wii_h, wii_l, wif_h, wif_l, wig_h, wig_l, wio_h, wio_l,
                   whi_h, whi_l, whf_h, whf_l, whg_h, whg_l, who_h, who_l,
                   bli_ref, blf_ref, blg_ref, blo_ref,
                   wp_h, wp_l, wo_h, wo_l, bout_ref,
                   lab_ref, sc_ref):
    iota_v = jax.lax.broadcasted_iota(jnp.int32, (_B, _V), 1)
    iota_c = jax.lax.broadcasted_iota(jnp.int32, (_B, _TCH), 1)
    lens = lens_ref[...][:, :1]

    def step(chunk):
        def body(tt, carry):
            h, c, lbl, labacc, scacc = carry
            t = chunk * _TCH + tt

            onehot = (iota_v == lbl).astype(_bf16)
            emb_hi = jnp.dot(onehot, ehi_ref[...],
                             preferred_element_type=_f32).astype(_bf16)
            emb_lo = jnp.dot(onehot, elo_ref[...],
                             preferred_element_type=_f32).astype(_bf16)
            h_hi, h_lo = _xsplit(h)

            def gate(wi_h, wi_l, wh_h, wh_l, b_ref):
                return (_dot3(emb_hi, emb_lo, wi_h, wi_l)
                        + _dot3(h_hi, h_lo, wh_h, wh_l)
                        + b_ref[...])

            g_i = gate(wii_h, wii_l, whi_h, whi_l, bli_ref)
            g_f = gate(wif_h, wif_l, whf_h, whf_l, blf_ref)
            g_g = gate(wig_h, wig_l, whg_h, whg_l, blg_ref)
            g_o = gate(wio_h, wio_l, who_h, who_l, blo_ref)
            c_new = (jax.nn.sigmoid(g_f) * c
                     + jax.nn.sigmoid(g_i) * jnp.tanh(g_g))
            h_new = jax.nn.sigmoid(g_o) * jnp.tanh(c_new)

            hn_hi, hn_lo = _xsplit(h_new)
            pre = encp_ref[t] + _dot3(hn_hi, hn_lo, wp_h, wp_l)
            jt_hi, jt_lo = _xsplit(jnp.tanh(pre))
            logits = _dot3(jt_hi, jt_lo, wo_h, wo_l) + bout_ref[...]

            m = jnp.max(logits, axis=1, keepdims=True)
            sym = jnp.min(jnp.where(logits == m, iota_v, _V),
                          axis=1, keepdims=True)
            score = -jnp.log(jnp.sum(jnp.exp(logits - m),
                                     axis=1, keepdims=True))

            blank = jnp.logical_or(sym == _BLANK, t >= lens)
            h = jnp.where(blank, h, h_new)
            c = jnp.where(blank, c, c_new)
            lbl = jnp.where(blank, lbl, sym)
            emit = jnp.where(blank, _BLANK, sym)

            colmask = iota_c == tt
            labacc = jnp.where(colmask,
                               jnp.broadcast_to(emit, (_B, _TCH)), labacc)
            scacc = jnp.where(colmask,
                              jnp.broadcast_to(score, (_B, _TCH)), scacc)
            return h, c, lbl, labacc, scacc
        return body

    h = jnp.zeros((_B, _DP), _f32)
    c = jnp.zeros((_B, _DP), _f32)
    lbl = jnp.full((_B, 1), _BLANK, jnp.int32)
    for chunk in range(_T // _TCH):
        init = (h, c, lbl,
                jnp.zeros((_B, _TCH), jnp.int32),
                jnp.zeros((_B, _TCH), _f32))
        h, c, lbl, labacc, scacc = jax.lax.fori_loop(
            0, _TCH, step(chunk), init)
        lab_ref[:, chunk * _TCH:(chunk + 1) * _TCH] = labacc
        sc_ref[:, chunk * _TCH:(chunk + 1) * _TCH] = scacc


def _full(shape):
    return pl.BlockSpec(shape, lambda i: (0,) * len(shape))


@jax.jit
def kernel(encoded_outs, encoded_lens, embed, W_ih, W_hh, b_lstm,
           W_enc, W_pred, b_joint, W_out, b_out):
    enc_flat = jnp.transpose(encoded_outs, (1, 0, 2)).reshape(_T * _B, _DE)

    encp = pl.pallas_call(
        _proj_kernel,
        grid=(_T * _B // _MB,),
        in_specs=[
            pl.BlockSpec((_MB, _DE), lambda i: (i, 0)),
            pl.BlockSpec((_DE, _DJ), lambda i: (0, 0)),
            pl.BlockSpec((1, _DJ), lambda i: (0, 0)),
        ],
        out_specs=pl.BlockSpec((_MB, _DJ), lambda i: (i, 0)),
        out_shape=jax.ShapeDtypeStruct((_T * _B, _DJ), _f32),
    )(enc_flat, W_enc, b_joint[None, :])
    encp = encp.reshape(_T, _B, _DJ)

    lens_b = jnp.broadcast_to(encoded_lens.astype(jnp.int32)[:, None],
                              (_B, 128))

    def wsplit(w):
        hi = w.astype(_bf16)
        lo = (w - hi.astype(_f32)).astype(_bf16)
        return hi, lo

    ehi, elo = wsplit(embed)
    wih = [wsplit(W_ih[:, k * _DP:(k + 1) * _DP]) for k in range(4)]
    whh = [wsplit(W_hh[:, k * _DP:(k + 1) * _DP]) for k in range(4)]
    bls = [b_lstm[None, k * _DP:(k + 1) * _DP] for k in range(4)]
    wp_h, wp_l = wsplit(W_pred)
    wo_h, wo_l = wsplit(W_out)

    bf_full = lambda shape: _full(shape)
    labels, scores = pl.pallas_call(
        _decode_kernel,
        grid=(1,),
        in_specs=(
            [_full((_T, _B, _DJ)), _full((_B, 128)),
             _full((_V, _DP)), _full((_V, _DP))]
            + [_full((_DP, _DP))] * 16
            + [_full((1, _DP))] * 4
            + [_full((_DP, _DJ))] * 2
            + [_full((_DJ, _V))] * 2
            + [_full((1, _V))]
        ),
        out_specs=[
            _full((_B, _T)),
            _full((_B, _T)),
        ],
        out_shape=[
            jax.ShapeDtypeStruct((_B, _T), jnp.int32),
            jax.ShapeDtypeStruct((_B, _T), _f32),
        ],
        compiler_params=pltpu.CompilerParams(
            dimension_semantics=("arbitrary",)),
    )(encp, lens_b, ehi, elo,
      wih[0][0], wih[0][1], wih[1][0], wih[1][1],
      wih[2][0], wih[2][1], wih[3][0], wih[3][1],
      whh[0][0], whh[0][1], whh[1][0], whh[1][1],
      whh[2][0], whh[2][1], whh[3][0], whh[3][1],
      bls[0], bls[1], bls[2], bls[3],
      wp_h, wp_l, wo_h, wo_l, b_out[None, :])
    return labels, scores


# speedup vs baseline: 1.7829x; 1.7829x over previous
import jax
import jax.numpy as jnp
from jax.experimental import pallas as pl
from jax.experimental.pallas import tpu as pltpu

_B = 16
_T = 512
_DE = 512
_DP = 320
_DJ = 320
_V = 1024
_BLANK = 0
_TCH = 128
_MB = 1024

_f32 = jnp.float32


def _proj_kernel(enc_ref, wenc_ref, bj_ref, out_ref):
    out_ref[...] = (jnp.dot(enc_ref[...], wenc_ref[...],
                            preferred_element_type=_f32)
                    + bj_ref[...])


def _decode_kernel(encp_ref, lens_ref, embed_ref,
                   wih_ref, whh_ref, bl_ref,
                   wpred_ref, wout_ref, bout_ref,
                   lab_ref, sc_ref):
    iota_v = jax.lax.broadcasted_iota(jnp.int32, (_B, _V), 1)
    iota_c = jax.lax.broadcasted_iota(jnp.int32, (_B, _TCH), 1)
    lens = lens_ref[...][:, :1]

    def step(chunk):
        def body(tt, carry):
            h, c, lbl, labacc, scacc = carry
            t = chunk * _TCH + tt

            onehot = (iota_v == lbl).astype(_f32)
            emb = jnp.dot(onehot, embed_ref[...],
                          preferred_element_type=_f32)

            gates = (jnp.dot(emb, wih_ref[...], preferred_element_type=_f32)
                     + jnp.dot(h, whh_ref[...], preferred_element_type=_f32)
                     + bl_ref[...])
            g_i = gates[:, 0:_DP]
            g_f = gates[:, _DP:2 * _DP]
            g_g = gates[:, 2 * _DP:3 * _DP]
            g_o = gates[:, 3 * _DP:4 * _DP]
            c_new = (jax.nn.sigmoid(g_f) * c
                     + jax.nn.sigmoid(g_i) * jnp.tanh(g_g))
            h_new = jax.nn.sigmoid(g_o) * jnp.tanh(c_new)

            pre = encp_ref[t] + jnp.dot(h_new, wpred_ref[...],
                                        preferred_element_type=_f32)
            logits = (jnp.dot(jnp.tanh(pre), wout_ref[...],
                              preferred_element_type=_f32)
                      + bout_ref[...])

            m = jnp.max(logits, axis=1, keepdims=True)
            sym = jnp.min(jnp.where(logits == m, iota_v, _V),
                          axis=1, keepdims=True)
            score = -jnp.log(jnp.sum(jnp.exp(logits - m),
                                     axis=1, keepdims=True))

            blank = jnp.logical_or(sym == _BLANK, t >= lens)
            h = jnp.where(blank, h, h_new)
            c = jnp.where(blank, c, c_new)
            lbl = jnp.where(blank, lbl, sym)
            emit = jnp.where(blank, _BLANK, sym)

            colmask = iota_c == tt
            labacc = jnp.where(colmask,
                               jnp.broadcast_to(emit, (_B, _TCH)), labacc)
            scacc = jnp.where(colmask,
                              jnp.broadcast_to(score, (_B, _TCH)), scacc)
            return h, c, lbl, labacc, scacc
        return body

    h = jnp.zeros((_B, _DP), _f32)
    c = jnp.zeros((_B, _DP), _f32)
    lbl = jnp.full((_B, 1), _BLANK, jnp.int32)
    for chunk in range(_T // _TCH):
        init = (h, c, lbl,
                jnp.zeros((_B, _TCH), jnp.int32),
                jnp.zeros((_B, _TCH), _f32))
        h, c, lbl, labacc, scacc = jax.lax.fori_loop(
            0, _TCH, step(chunk), init, unroll=2)
        lab_ref[:, chunk * _TCH:(chunk + 1) * _TCH] = labacc
        sc_ref[:, chunk * _TCH:(chunk + 1) * _TCH] = scacc


def _full(shape):
    return pl.BlockSpec(shape, lambda i: (0,) * len(shape))


@jax.jit
def kernel(encoded_outs, encoded_lens, embed, W_ih, W_hh, b_lstm,
           W_enc, W_pred, b_joint, W_out, b_out):
    enc_flat = jnp.transpose(encoded_outs, (1, 0, 2)).reshape(_T * _B, _DE)

    encp = pl.pallas_call(
        _proj_kernel,
        grid=(_T * _B // _MB,),
        in_specs=[
            pl.BlockSpec((_MB, _DE), lambda i: (i, 0)),
            pl.BlockSpec((_DE, _DJ), lambda i: (0, 0)),
            pl.BlockSpec((1, _DJ), lambda i: (0, 0)),
        ],
        out_specs=pl.BlockSpec((_MB, _DJ), lambda i: (i, 0)),
        out_shape=jax.ShapeDtypeStruct((_T * _B, _DJ), _f32),
    )(enc_flat, W_enc, b_joint[None, :])
    encp = encp.reshape(_T, _B, _DJ)

    lens_b = jnp.broadcast_to(encoded_lens.astype(jnp.int32)[:, None],
                              (_B, 128))

    labels, scores = pl.pallas_call(
        _decode_kernel,
        grid=(1,),
        in_specs=[
            _full((_T, _B, _DJ)),
            _full((_B, 128)),
            _full((_V, _DP)),
            _full((_DP, 4 * _DP)),
            _full((_DP, 4 * _DP)),
            _full((1, 4 * _DP)),
            _full((_DP, _DJ)),
            _full((_DJ, _V)),
            _full((1, _V)),
        ],
        out_specs=[
            _full((_B, _T)),
            _full((_B, _T)),
        ],
        out_shape=[
            jax.ShapeDtypeStruct((_B, _T), jnp.int32),
            jax.ShapeDtypeStruct((_B, _T), _f32),
        ],
        compiler_params=pltpu.CompilerParams(
            dimension_semantics=("arbitrary",)),
    )(encp, lens_b, embed, W_ih, W_hh, b_lstm[None, :],
      W_pred, W_out, b_out[None, :])
    return labels, scores


# speedup vs baseline: 1.8100x; 1.0152x over previous
import jax
import jax.numpy as jnp
from jax.experimental import pallas as pl
from jax.experimental.pallas import tpu as pltpu

_B = 16
_T = 512
_DE = 512
_DP = 320
_DJ = 320
_V = 1024
_BLANK = 0
_TCH = 128
_MB = 1024

_f32 = jnp.float32


def _proj_kernel(enc_ref, wenc_ref, bj_ref, out_ref):
    out_ref[...] = (jnp.dot(enc_ref[...], wenc_ref[...],
                            preferred_element_type=_f32)
                    + bj_ref[...])


def _decode_kernel(encp_ref, lens_ref, embed_ref,
                   wih_ref, whh_ref, bl_ref,
                   wpred_ref, wout_ref, bout_ref,
                   lab_ref, sc_ref):
    iota_v = jax.lax.broadcasted_iota(jnp.int32, (_B, _V), 1)
    iota_c = jax.lax.broadcasted_iota(jnp.int32, (_B, _TCH), 1)
    lens = lens_ref[...][:, :1]

    def step(chunk):
        def body(tt, carry):
            h, c, lbl, labacc, scacc = carry
            t = chunk * _TCH + tt

            onehot = (iota_v == lbl).astype(_f32)
            emb = jnp.dot(onehot, embed_ref[...],
                          preferred_element_type=_f32)

            gates = (jnp.dot(emb, wih_ref[...], preferred_element_type=_f32)
                     + jnp.dot(h, whh_ref[...], preferred_element_type=_f32)
                     + bl_ref[...])
            g_i = gates[:, 0:_DP]
            g_f = gates[:, _DP:2 * _DP]
            g_g = gates[:, 2 * _DP:3 * _DP]
            g_o = gates[:, 3 * _DP:4 * _DP]
            c_new = (jax.nn.sigmoid(g_f) * c
                     + jax.nn.sigmoid(g_i) * jnp.tanh(g_g))
            h_new = jax.nn.sigmoid(g_o) * jnp.tanh(c_new)

            pre = encp_ref[t] + jnp.dot(h_new, wpred_ref[...],
                                        preferred_element_type=_f32)
            logits = (jnp.dot(jnp.tanh(pre), wout_ref[...],
                              preferred_element_type=_f32)
                      + bout_ref[...])

            m = jnp.max(logits, axis=1, keepdims=True)
            sym = jnp.min(jnp.where(logits == m, iota_v, _V),
                          axis=1, keepdims=True)
            score = -jnp.log(jnp.sum(jnp.exp(logits - m),
                                     axis=1, keepdims=True))

            blank = jnp.logical_or(sym == _BLANK, t >= lens)
            h = jnp.where(blank, h, h_new)
            c = jnp.where(blank, c, c_new)
            lbl = jnp.where(blank, lbl, sym)
            emit = jnp.where(blank, _BLANK, sym)

            colmask = iota_c == tt
            labacc = jnp.where(colmask,
                               jnp.broadcast_to(emit, (_B, _TCH)), labacc)
            scacc = jnp.where(colmask,
                              jnp.broadcast_to(score, (_B, _TCH)), scacc)
            return h, c, lbl, labacc, scacc
        return body

    h = jnp.zeros((_B, _DP), _f32)
    c = jnp.zeros((_B, _DP), _f32)
    lbl = jnp.full((_B, 1), _BLANK, jnp.int32)
    for chunk in range(_T // _TCH):
        init = (h, c, lbl,
                jnp.zeros((_B, _TCH), jnp.int32),
                jnp.zeros((_B, _TCH), _f32))
        h, c, lbl, labacc, scacc = jax.lax.fori_loop(
            0, _TCH, step(chunk), init, unroll=4)
        lab_ref[:, chunk * _TCH:(chunk + 1) * _TCH] = labacc
        sc_ref[:, chunk * _TCH:(chunk + 1) * _TCH] = scacc


def _full(shape):
    return pl.BlockSpec(shape, lambda i: (0,) * len(shape))


@jax.jit
def kernel(encoded_outs, encoded_lens, embed, W_ih, W_hh, b_lstm,
           W_enc, W_pred, b_joint, W_out, b_out):
    enc_flat = jnp.transpose(encoded_outs, (1, 0, 2)).reshape(_T * _B, _DE)

    encp = pl.pallas_call(
        _proj_kernel,
        grid=(_T * _B // _MB,),
        in_specs=[
            pl.BlockSpec((_MB, _DE), lambda i: (i, 0)),
            pl.BlockSpec((_DE, _DJ), lambda i: (0, 0)),
            pl.BlockSpec((1, _DJ), lambda i: (0, 0)),
        ],
        out_specs=pl.BlockSpec((_MB, _DJ), lambda i: (i, 0)),
        out_shape=jax.ShapeDtypeStruct((_T * _B, _DJ), _f32),
    )(enc_flat, W_enc, b_joint[None, :])
    encp = encp.reshape(_T, _B, _DJ)

    lens_b = jnp.broadcast_to(encoded_lens.astype(jnp.int32)[:, None],
                              (_B, 128))

    labels, scores = pl.pallas_call(
        _decode_kernel,
        grid=(1,),
        in_specs=[
            _full((_T, _B, _DJ)),
            _full((_B, 128)),
            _full((_V, _DP)),
            _full((_DP, 4 * _DP)),
            _full((_DP, 4 * _DP)),
            _full((1, 4 * _DP)),
            _full((_DP, _DJ)),
            _full((_DJ, _V)),
            _full((1, _V)),
        ],
        out_specs=[
            _full((_B, _T)),
            _full((_B, _T)),
        ],
        out_shape=[
            jax.ShapeDtypeStruct((_B, _T), jnp.int32),
            jax.ShapeDtypeStruct((_B, _T), _f32),
        ],
        compiler_params=pltpu.CompilerParams(
            dimension_semantics=("arbitrary",)),
    )(encp, lens_b, embed, W_ih, W_hh, b_lstm[None, :],
      W_pred, W_out, b_out[None, :])
    return labels, scores


# speedup vs baseline: 1.8297x; 1.0109x over previous
import jax
import jax.numpy as jnp
from jax.experimental import pallas as pl
from jax.experimental.pallas import tpu as pltpu

_B = 16
_T = 512
_DE = 512
_DP = 320
_DJ = 320
_V = 1024
_BLANK = 0
_TCH = 128
_MB = 1024

_f32 = jnp.float32


def _proj_kernel(enc_ref, wenc_ref, bj_ref, out_ref):
    out_ref[...] = (jnp.dot(enc_ref[...], wenc_ref[...],
                            preferred_element_type=_f32)
                    + bj_ref[...])


def _decode_kernel(encp_ref, lens_ref, embed_ref,
                   wih_ref, whh_ref, bl_ref,
                   wpred_ref, wout_ref, bout_ref,
                   lab_ref, sc_ref):
    iota_v = jax.lax.broadcasted_iota(jnp.int32, (_B, _V), 1)
    iota_c = jax.lax.broadcasted_iota(jnp.int32, (_B, _TCH), 1)
    lens = lens_ref[...][:, :1]

    def step(chunk):
        def body(tt, carry):
            h, c, lbl, labacc, scacc = carry
            t = chunk * _TCH + tt

            onehot = (iota_v == lbl).astype(_f32)
            emb = jnp.dot(onehot, embed_ref[...],
                          preferred_element_type=_f32)

            gates = (jnp.dot(emb, wih_ref[...], preferred_element_type=_f32)
                     + jnp.dot(h, whh_ref[...], preferred_element_type=_f32)
                     + bl_ref[...])
            g_i = gates[:, 0:_DP]
            g_f = gates[:, _DP:2 * _DP]
            g_g = gates[:, 2 * _DP:3 * _DP]
            g_o = gates[:, 3 * _DP:4 * _DP]
            c_new = (jax.nn.sigmoid(g_f) * c
                     + jax.nn.sigmoid(g_i) * jnp.tanh(g_g))
            h_new = jax.nn.sigmoid(g_o) * jnp.tanh(c_new)

            pre = encp_ref[t] + jnp.dot(h_new, wpred_ref[...],
                                        preferred_element_type=_f32)
            logits = (jnp.dot(jnp.tanh(pre), wout_ref[...],
                              preferred_element_type=_f32)
                      + bout_ref[...])

            m = jnp.max(logits, axis=1, keepdims=True)
            sym = jnp.min(jnp.where(logits == m, iota_v, _V),
                          axis=1, keepdims=True)
            score = -jnp.log(jnp.sum(jnp.exp(logits - m),
                                     axis=1, keepdims=True))

            blank = jnp.logical_or(sym == _BLANK, t >= lens)
            h = jnp.where(blank, h, h_new)
            c = jnp.where(blank, c, c_new)
            lbl = jnp.where(blank, lbl, sym)
            emit = jnp.where(blank, _BLANK, sym)

            colmask = iota_c == tt
            labacc = jnp.where(colmask,
                               jnp.broadcast_to(emit, (_B, _TCH)), labacc)
            scacc = jnp.where(colmask,
                              jnp.broadcast_to(score, (_B, _TCH)), scacc)
            return h, c, lbl, labacc, scacc
        return body

    h = jnp.zeros((_B, _DP), _f32)
    c = jnp.zeros((_B, _DP), _f32)
    lbl = jnp.full((_B, 1), _BLANK, jnp.int32)
    for chunk in range(_T // _TCH):
        init = (h, c, lbl,
                jnp.zeros((_B, _TCH), jnp.int32),
                jnp.zeros((_B, _TCH), _f32))
        h, c, lbl, labacc, scacc = jax.lax.fori_loop(
            0, _TCH, step(chunk), init, unroll=8)
        lab_ref[:, chunk * _TCH:(chunk + 1) * _TCH] = labacc
        sc_ref[:, chunk * _TCH:(chunk + 1) * _TCH] = scacc


def _full(shape):
    return pl.BlockSpec(shape, lambda i: (0,) * len(shape))


@jax.jit
def kernel(encoded_outs, encoded_lens, embed, W_ih, W_hh, b_lstm,
           W_enc, W_pred, b_joint, W_out, b_out):
    enc_flat = jnp.transpose(encoded_outs, (1, 0, 2)).reshape(_T * _B, _DE)

    encp = pl.pallas_call(
        _proj_kernel,
        grid=(_T * _B // _MB,),
        in_specs=[
            pl.BlockSpec((_MB, _DE), lambda i: (i, 0)),
            pl.BlockSpec((_DE, _DJ), lambda i: (0, 0)),
            pl.BlockSpec((1, _DJ), lambda i: (0, 0)),
        ],
        out_specs=pl.BlockSpec((_MB, _DJ), lambda i: (i, 0)),
        out_shape=jax.ShapeDtypeStruct((_T * _B, _DJ), _f32),
    )(enc_flat, W_enc, b_joint[None, :])
    encp = encp.reshape(_T, _B, _DJ)

    lens_b = jnp.broadcast_to(encoded_lens.astype(jnp.int32)[:, None],
                              (_B, 128))

    labels, scores = pl.pallas_call(
        _decode_kernel,
        grid=(1,),
        in_specs=[
            _full((_T, _B, _DJ)),
            _full((_B, 128)),
            _full((_V, _DP)),
            _full((_DP, 4 * _DP)),
            _full((_DP, 4 * _DP)),
            _full((1, 4 * _DP)),
            _full((_DP, _DJ)),
            _full((_DJ, _V)),
            _full((1, _V)),
        ],
        out_specs=[
            _full((_B, _T)),
            _full((_B, _T)),
        ],
        out_shape=[
            jax.ShapeDtypeStruct((_B, _T), jnp.int32),
            jax.ShapeDtypeStruct((_B, _T), _f32),
        ],
        compiler_params=pltpu.CompilerParams(
            dimension_semantics=("arbitrary",)),
    )(encp, lens_b, embed, W_ih, W_hh, b_lstm[None, :],
      W_pred, W_out, b_out[None, :])
    return labels, scores
